# baseline (device time: 41681 ns/iter reference)
import jax
import jax.numpy as jnp
from jax import lax
from jax.experimental import pallas as pl
from jax.experimental.pallas import tpu as pltpu

N_DEV = 4
M_PER = 1024
K = 4096
N = 2048
N_PER = N // N_DEV
N_BLK = N_PER // 2
N_STEPS = 2 * N_DEV
N_SENDS = 2 * (N_DEV - 1)


def kernel(x, w_mat, scale_x, scale_w):
    def body(x_ref, w_ref, sx_ref, sw_ref, out_ref,
             xq, send_buf, recv_buf, ssc_buf, rsc_buf,
             send_sems, recv_sems, ssc_sems, rsc_sems):
        h = pl.program_id(0)
        my = lax.axis_index("i")

        @pl.when(h == 0)
        def _():
            barrier_sem = pltpu.get_barrier_semaphore()
            for off in range(1, N_DEV):
                peer = lax.rem(my + off, N_DEV)
                pl.semaphore_signal(barrier_sem, inc=1, device_id=(peer,),
                                    device_id_type=pl.DeviceIdType.MESH)
            pl.semaphore_wait(barrier_sem, N_DEV - 1)
            xq[...] = x_ref[...].astype(jnp.float8_e4m3fn)

        acc = jnp.dot(xq[...], w_ref[...].astype(jnp.float8_e4m3fn),
                      preferred_element_type=jnp.float32)
        s = sx_ref[0] * sw_ref[0]
        v = acc * s
        y = v * jax.nn.sigmoid(v)

        def peer_of(q):
            return lax.rem(my + q // 2 + 1, N_DEV)

        def data_desc(q):
            return pltpu.make_async_remote_copy(
                src_ref=send_buf.at[q], dst_ref=recv_buf.at[q],
                send_sem=send_sems.at[q], recv_sem=recv_sems.at[q],
                device_id=(peer_of(q),),
                device_id_type=pl.DeviceIdType.MESH,
            )

        def scale_desc(q):
            return pltpu.make_async_remote_copy(
                src_ref=ssc_buf.at[q], dst_ref=rsc_buf.at[q],
                send_sem=ssc_sems.at[q], recv_sem=rsc_sems.at[q],
                device_id=(peer_of(q),),
                device_id_type=pl.DeviceIdType.MESH,
            )

        for q in range(N_SENDS):
            @pl.when(h == q)
            def _(q=q):
                m = jnp.maximum(jnp.max(jnp.abs(y)), 1e-20)
                send_buf[q] = jnp.clip(
                    jnp.round(y * (127.0 / m)), -127.0, 127.0
                ).astype(jnp.int8)
                ssc_buf[q] = jnp.full((8, 128), m * (1.0 / 127.0),
                                      jnp.float32)
                scale_desc(q).start()
                data_desc(q).start()

        for q in range(N_SENDS, N_STEPS):
            @pl.when(h == q)
            def _(q=q):
                half = q % 2
                out_ref[pl.ds(my * M_PER, M_PER),
                        pl.ds(half * N_BLK, N_BLK)] = y

        for q in range(N_SENDS):
            @pl.when(h == q + 2)
            def _(q=q):
                off = q // 2 + 1
                half = q % 2
                src = lax.rem(my - off + N_DEV, N_DEV)
                scale_desc(q).wait_recv()
                data_desc(q).wait_recv()
                sc = jnp.max(rsc_buf[q])
                out_ref[pl.ds(src * M_PER, M_PER),
                        pl.ds(half * N_BLK, N_BLK)] = (
                    recv_buf[q].astype(jnp.float32) * sc)

        @pl.when(h == N_STEPS - 1)
        def _():
            for q in range(N_SENDS):
                data_desc(q).wait_send()
                scale_desc(q).wait_send()

    return pl.pallas_call(
        body,
        grid=(N_STEPS,),
        out_shape=jax.ShapeDtypeStruct((N_DEV * M_PER, N_PER), jnp.float32),
        in_specs=[
            pl.BlockSpec((M_PER, K), lambda h: (0, 0),
                         memory_space=pltpu.VMEM),
            pl.BlockSpec(
                (K, N_BLK),
                lambda h: (0, 2 * lax.rem(lax.axis_index("i") + h // 2 + 1,
                                          N_DEV) + h % 2),
                memory_space=pltpu.VMEM),
            pl.BlockSpec(memory_space=pltpu.SMEM),
            pl.BlockSpec(memory_space=pltpu.SMEM),
        ],
        out_specs=pl.BlockSpec((N_DEV * M_PER, N_PER), lambda h: (0, 0),
                               memory_space=pltpu.VMEM),
        scratch_shapes=[
            pltpu.VMEM((M_PER, K), jnp.float8_e4m3fn),
            pltpu.VMEM((N_SENDS, M_PER, N_BLK), jnp.int8),
            pltpu.VMEM((N_SENDS, M_PER, N_BLK), jnp.int8),
            pltpu.VMEM((N_SENDS, 8, 128), jnp.float32),
            pltpu.VMEM((N_SENDS, 8, 128), jnp.float32),
            pltpu.SemaphoreType.DMA((N_SENDS,)),
            pltpu.SemaphoreType.DMA((N_SENDS,)),
            pltpu.SemaphoreType.DMA((N_SENDS,)),
            pltpu.SemaphoreType.DMA((N_SENDS,)),
        ],
        compiler_params=pltpu.CompilerParams(
            collective_id=0,
            dimension_semantics=("arbitrary",),
            vmem_limit_bytes=100 * 1024 * 1024,
        ),
    )(x, w_mat, scale_x, scale_w)


# device time: 40979 ns/iter; 1.0171x vs baseline; 1.0171x over previous
import jax
import jax.numpy as jnp
from jax import lax
from jax.experimental import pallas as pl
from jax.experimental.pallas import tpu as pltpu

N_DEV = 4
M_PER = 1024
K = 4096
N = 2048
N_PER = N // N_DEV
N_BLK = N_PER // 2
N_STEPS = 2 * N_DEV
N_SENDS = 2 * (N_DEV - 1)

OFFS = (2, 2, 1, 3, 1, 3, 0, 0)
HALF = (0, 1, 0, 0, 1, 1, 0, 1)


def kernel(x, w_mat, scale_x, scale_w):
    def body(x_ref, w_ref, sx_ref, sw_ref, out_ref,
             xq, send_buf, recv_buf, ssc_buf, rsc_buf,
             send_sems, recv_sems, ssc_sems, rsc_sems):
        h = pl.program_id(0)
        my = lax.axis_index("i")

        @pl.when(h == 0)
        def _():
            barrier_sem = pltpu.get_barrier_semaphore()
            for off in range(1, N_DEV):
                peer = lax.rem(my + off, N_DEV)
                pl.semaphore_signal(barrier_sem, inc=1, device_id=(peer,),
                                    device_id_type=pl.DeviceIdType.MESH)
            pl.semaphore_wait(barrier_sem, N_DEV - 1)
            xq[...] = x_ref[...].astype(jnp.float8_e4m3fn)

        acc = jnp.dot(xq[...], w_ref[...].astype(jnp.float8_e4m3fn),
                      preferred_element_type=jnp.float32)
        s = sx_ref[0] * sw_ref[0]
        v = acc * s
        y = v * jax.nn.sigmoid(v)

        def peer_of(q):
            return lax.rem(my + OFFS[q], N_DEV)

        def data_desc(q):
            return pltpu.make_async_remote_copy(
                src_ref=send_buf.at[q], dst_ref=recv_buf.at[q],
                send_sem=send_sems.at[q], recv_sem=recv_sems.at[q],
                device_id=(peer_of(q),),
                device_id_type=pl.DeviceIdType.MESH,
            )

        def scale_desc(q):
            return pltpu.make_async_remote_copy(
                src_ref=ssc_buf.at[q], dst_ref=rsc_buf.at[q],
                send_sem=ssc_sems.at[q], recv_sem=rsc_sems.at[q],
                device_id=(peer_of(q),),
                device_id_type=pl.DeviceIdType.MESH,
            )

        for q in range(N_SENDS):
            @pl.when(h == q)
            def _(q=q):
                m = jnp.maximum(jnp.max(jnp.abs(y)), 1e-20)
                send_buf[q] = jnp.clip(
                    jnp.round(y * (127.0 / m)), -127.0, 127.0
                ).astype(jnp.int8)
                ssc_buf[q] = jnp.full((8, 128), m * (1.0 / 127.0),
                                      jnp.float32)
                scale_desc(q).start()
                data_desc(q).start()

        for q in range(N_SENDS, N_STEPS):
            @pl.when(h == q)
            def _(q=q):
                half = HALF[q]
                out_ref[pl.ds(my * M_PER, M_PER),
                        pl.ds(half * N_BLK, N_BLK)] = y

        for q in range(N_SENDS):
            @pl.when(h == q + 2)
            def _(q=q):
                off = OFFS[q]
                half = HALF[q]
                src = lax.rem(my - off + N_DEV, N_DEV)
                scale_desc(q).wait_recv()
                data_desc(q).wait_recv()
                sc = jnp.max(rsc_buf[q])
                out_ref[pl.ds(src * M_PER, M_PER),
                        pl.ds(half * N_BLK, N_BLK)] = (
                    recv_buf[q].astype(jnp.float32) * sc)

        @pl.when(h == N_STEPS - 1)
        def _():
            for q in range(N_SENDS):
                data_desc(q).wait_send()
                scale_desc(q).wait_send()

    return pl.pallas_call(
        body,
        grid=(N_STEPS,),
        out_shape=jax.ShapeDtypeStruct((N_DEV * M_PER, N_PER), jnp.float32),
        in_specs=[
            pl.BlockSpec((M_PER, K), lambda h: (0, 0),
                         memory_space=pltpu.VMEM),
            pl.BlockSpec(
                (K, N_BLK),
                lambda h: (0, 2 * lax.rem(
                    lax.axis_index("i") + jnp.where(
                        h < 2, 2,
                        jnp.where(h < 6,
                                  jnp.where(h % 2 == 0, 1, 3), 0)),
                    N_DEV) + jnp.where(
                        h < 2, h % 2,
                        jnp.where(h < 6, jnp.where(h < 4, 0, 1), h % 2))),
                memory_space=pltpu.VMEM),
            pl.BlockSpec(memory_space=pltpu.SMEM),
            pl.BlockSpec(memory_space=pltpu.SMEM),
        ],
        out_specs=pl.BlockSpec((N_DEV * M_PER, N_PER), lambda h: (0, 0),
                               memory_space=pltpu.VMEM),
        scratch_shapes=[
            pltpu.VMEM((M_PER, K), jnp.float8_e4m3fn),
            pltpu.VMEM((N_SENDS, M_PER, N_BLK), jnp.int8),
            pltpu.VMEM((N_SENDS, M_PER, N_BLK), jnp.int8),
            pltpu.VMEM((N_SENDS, 8, 128), jnp.float32),
            pltpu.VMEM((N_SENDS, 8, 128), jnp.float32),
            pltpu.SemaphoreType.DMA((N_SENDS,)),
            pltpu.SemaphoreType.DMA((N_SENDS,)),
            pltpu.SemaphoreType.DMA((N_SENDS,)),
            pltpu.SemaphoreType.DMA((N_SENDS,)),
        ],
        compiler_params=pltpu.CompilerParams(
            collective_id=0,
            dimension_semantics=("arbitrary",),
            vmem_limit_bytes=100 * 1024 * 1024,
        ),
    )(x, w_mat, scale_x, scale_w)
